# Initial kernel scaffold; baseline (speedup 1.0000x reference)
import jax
import jax.numpy as jnp
from jax.experimental import pallas as pl


def kernel(x, edge_index, W1, b1, W2, b2, W_fc, b_fc):
    raise NotImplementedError("write your pallas kernel here")


# speedup vs baseline: 23.9370x; 23.9370x over previous
import functools

import jax
import jax.numpy as jnp
from jax import lax
from jax.experimental import pallas as pl
from jax.experimental.pallas import tpu as pltpu
from jax.experimental.pallas import tpu_sc as plsc

N_NODES = 10000
N_EDGES = 320000
IN_CH = 128
HID = 64

NC = 2
NS = 16
NW = NC * NS
EPW = N_EDGES // NW
CHUNK = 80
NCHUNK = EPW // CHUNK
ROWS_PT = N_NODES // NS
ZROWS = 125

_mesh = plsc.VectorSubcoreMesh(core_axis_name="c", subcore_axis_name="s")
_sc_params = pltpu.CompilerParams(use_tc_tiling_on_sc=False)


DEGW = 16


@functools.partial(
    pl.kernel,
    out_type=jax.ShapeDtypeStruct((NC, NS, ROWS_PT, DEGW), jnp.float32),
    mesh=_mesh,
    scratch_types=[
        pltpu.VMEM((NCHUNK, CHUNK), jnp.int32),
        pltpu.VMEM((CHUNK, DEGW), jnp.float32),
        pltpu.VMEM((ROWS_PT, DEGW), jnp.float32),
        pltpu.VMEM_SHARED((N_NODES, DEGW), jnp.float32),
    ],
    compiler_params=_sc_params,
)
def _deg_kernel(dst_hbm, ones_hbm, zeros_hbm, out_hbm, dstv, onesv, zbuf,
                deg_sh):
    c = lax.axis_index("c")
    s = lax.axis_index("s")
    pltpu.sync_copy(dst_hbm.at[c * NS + s], dstv)
    pltpu.sync_copy(ones_hbm, onesv)
    pltpu.sync_copy(zeros_hbm, zbuf)
    pltpu.sync_copy(zbuf, deg_sh.at[pl.ds(s * ROWS_PT, ROWS_PT)])
    plsc.subcore_barrier()

    def body(j, carry):
        pltpu.sync_copy(onesv, deg_sh.at[dstv.at[j]], add=True)
        return carry

    lax.fori_loop(0, NCHUNK, body, 0)
    plsc.subcore_barrier()
    pltpu.sync_copy(deg_sh.at[pl.ds(s * ROWS_PT, ROWS_PT)], out_hbm.at[c, s])


@functools.partial(
    pl.kernel,
    out_type=jax.ShapeDtypeStruct((NC, NS, ROWS_PT, HID), jnp.float32),
    mesh=_mesh,
    scratch_types=[
        pltpu.VMEM((NCHUNK, CHUNK), jnp.int32),
        pltpu.VMEM((NCHUNK, CHUNK), jnp.int32),
        pltpu.VMEM((CHUNK, HID), jnp.float32),
        pltpu.VMEM((ZROWS, HID), jnp.float32),
        pltpu.VMEM_SHARED((N_NODES, HID), jnp.float32),
        pltpu.SemaphoreType.DMA,
    ],
    compiler_params=_sc_params,
)
def _agg_kernel(gs_hbm, src_hbm, dst_hbm, zrows_hbm, out_hbm,
                srcv, dstv, rows, zbuf, acc_sh, sem):
    c = lax.axis_index("c")
    s = lax.axis_index("s")
    pltpu.sync_copy(src_hbm.at[c * NS + s], srcv)
    pltpu.sync_copy(dst_hbm.at[c * NS + s], dstv)
    pltpu.sync_copy(zrows_hbm, zbuf)
    for t in range(ROWS_PT // ZROWS):
        pltpu.sync_copy(zbuf, acc_sh.at[pl.ds(s * ROWS_PT + t * ZROWS, ZROWS)])
    plsc.subcore_barrier()

    def body(j, carry):
        pltpu.async_copy(gs_hbm.at[srcv.at[j]], rows, sem).wait()
        pltpu.sync_copy(rows, acc_sh.at[dstv.at[j]], add=True)
        return carry

    lax.fori_loop(0, NCHUNK, body, 0)
    plsc.subcore_barrier()
    pltpu.sync_copy(acc_sh.at[pl.ds(s * ROWS_PT, ROWS_PT)], out_hbm.at[c, s])


def _tc1_body(degs_ref, x_ref, w1_ref, gs_ref, dinv_ref):
    deg = degs_ref[0][:, :1] + degs_ref[1][:, :1] + 1.0
    dinv = lax.rsqrt(deg)
    dinv_ref[...] = dinv
    h = jnp.dot(x_ref[...], w1_ref[...], preferred_element_type=jnp.float32)
    gs_ref[...] = h * dinv


_tc1 = pl.pallas_call(
    _tc1_body,
    out_shape=(
        jax.ShapeDtypeStruct((N_NODES, HID), jnp.float32),
        jax.ShapeDtypeStruct((N_NODES, 1), jnp.float32),
    ),
)


def _tc2_body(accp_ref, gs1_ref, dinv_ref, b1_ref, w2_ref, out_ref):
    agg = accp_ref[0] + accp_ref[1] + gs1_ref[...]
    h = jnp.maximum(agg * dinv_ref[...] + b1_ref[...], 0.0)
    out_ref[...] = jnp.dot(
        h, w2_ref[...], preferred_element_type=jnp.float32) * dinv_ref[...]


_tc2 = pl.pallas_call(
    _tc2_body,
    out_shape=jax.ShapeDtypeStruct((N_NODES, HID), jnp.float32),
)


def _tc3_body(accp_ref, gs2_ref, dinv_ref, b2_ref, wfc_ref, bfc_ref, out_ref):
    agg = accp_ref[0] + accp_ref[1] + gs2_ref[...]
    h = jnp.maximum(agg * dinv_ref[...] + b2_ref[...], 0.0)
    out_ref[...] = jnp.dot(
        h, wfc_ref[...], preferred_element_type=jnp.float32) + bfc_ref[...]


_tc3 = pl.pallas_call(
    _tc3_body,
    out_shape=jax.ShapeDtypeStruct((N_NODES, 1), jnp.float32),
)


def kernel(x, edge_index, W1, b1, W2, b2, W_fc, b_fc):
    src = edge_index[0].reshape(NW, NCHUNK, CHUNK)
    dst = edge_index[1].reshape(NW, NCHUNK, CHUNK)
    zrows = jnp.zeros((ZROWS, HID), jnp.float32)
    ones_deg = jnp.ones((CHUNK, DEGW), jnp.float32)
    zeros_deg = jnp.zeros((ROWS_PT, DEGW), jnp.float32)

    degp = _deg_kernel(dst, ones_deg, zeros_deg).reshape(NC, N_NODES, DEGW)
    gs1, dinv = _tc1(degp, x, W1)
    acc1 = _agg_kernel(gs1, src, dst, zrows).reshape(NC, N_NODES, HID)
    gs2 = _tc2(acc1, gs1, dinv, b1.reshape(1, HID), W2)
    acc2 = _agg_kernel(gs2, src, dst, zrows).reshape(NC, N_NODES, HID)
    out = _tc3(acc2, gs2, dinv, b2.reshape(1, HID), W_fc, b_fc.reshape(1, 1))
    return out


# speedup vs baseline: 38.2524x; 1.5980x over previous
import functools

import jax
import jax.numpy as jnp
from jax import lax
from jax.experimental import pallas as pl
from jax.experimental.pallas import tpu as pltpu
from jax.experimental.pallas import tpu_sc as plsc

N_NODES = 10000
N_EDGES = 320000
IN_CH = 128
HID = 64

NC = 2
NS = 16
NW = NC * NS
EPW = N_EDGES // NW
CHUNK = 125
NCHUNK = EPW // CHUNK
ROWS_PT = N_NODES // NS
ZROWS = 125

_mesh = plsc.VectorSubcoreMesh(core_axis_name="c", subcore_axis_name="s")
_sc_params = pltpu.CompilerParams(use_tc_tiling_on_sc=False)


DEGW = 16


@functools.partial(
    pl.kernel,
    out_type=jax.ShapeDtypeStruct((NC, NS, ROWS_PT, DEGW), jnp.float32),
    mesh=_mesh,
    scratch_types=[
        pltpu.VMEM((NCHUNK, CHUNK), jnp.int32),
        pltpu.VMEM((CHUNK, DEGW), jnp.float32),
        pltpu.VMEM((ROWS_PT, DEGW), jnp.float32),
        pltpu.VMEM_SHARED((N_NODES, DEGW), jnp.float32),
    ],
    compiler_params=_sc_params,
)
def _deg_kernel(dst_hbm, ones_hbm, zeros_hbm, out_hbm, dstv, onesv, zbuf,
                deg_sh):
    c = lax.axis_index("c")
    s = lax.axis_index("s")
    pltpu.sync_copy(dst_hbm.at[c * NS + s], dstv)
    pltpu.sync_copy(ones_hbm, onesv)
    pltpu.sync_copy(zeros_hbm, zbuf)
    pltpu.sync_copy(zbuf, deg_sh.at[pl.ds(s * ROWS_PT, ROWS_PT)])
    plsc.subcore_barrier()

    def body(j, carry):
        pltpu.sync_copy(onesv, deg_sh.at[dstv.at[j]], add=True)
        return carry

    lax.fori_loop(0, NCHUNK, body, 0)
    plsc.subcore_barrier()
    pltpu.sync_copy(deg_sh.at[pl.ds(s * ROWS_PT, ROWS_PT)], out_hbm.at[c, s])


@functools.partial(
    pl.kernel,
    out_type=jax.ShapeDtypeStruct((NC, NS, ROWS_PT, HID), jnp.float32),
    mesh=_mesh,
    scratch_types=[
        pltpu.VMEM((NCHUNK, CHUNK), jnp.int32),
        pltpu.VMEM((NCHUNK, CHUNK), jnp.int32),
        pltpu.VMEM((CHUNK, HID), jnp.float32),
        pltpu.VMEM((CHUNK, HID), jnp.float32),
        pltpu.VMEM((ZROWS, HID), jnp.float32),
        pltpu.VMEM_SHARED((N_NODES, HID), jnp.float32),
        pltpu.SemaphoreType.DMA,
        pltpu.SemaphoreType.DMA,
    ],
    compiler_params=_sc_params,
)
def _agg_kernel(gs_hbm, src_hbm, dst_hbm, zrows_hbm, out_hbm,
                srcv, dstv, rows0, rows1, zbuf, acc_sh, sem0, sem1):
    c = lax.axis_index("c")
    s = lax.axis_index("s")
    pltpu.sync_copy(src_hbm.at[c * NS + s], srcv)
    pltpu.sync_copy(dst_hbm.at[c * NS + s], dstv)
    pltpu.sync_copy(zrows_hbm, zbuf)
    for t in range(ROWS_PT // ZROWS):
        pltpu.sync_copy(zbuf, acc_sh.at[pl.ds(s * ROWS_PT + t * ZROWS, ZROWS)])
    plsc.subcore_barrier()

    pltpu.async_copy(gs_hbm.at[srcv.at[0]], rows0, sem0)
    pltpu.async_copy(gs_hbm.at[srcv.at[1]], rows1, sem1)

    def body(i, carry):
        j = 2 * i
        pltpu.make_async_copy(gs_hbm.at[srcv.at[j]], rows0, sem0).wait()
        pltpu.sync_copy(rows0, acc_sh.at[dstv.at[j]], add=True)

        @pl.when(j + 2 < NCHUNK)
        def _():
            pltpu.async_copy(gs_hbm.at[srcv.at[j + 2]], rows0, sem0)

        pltpu.make_async_copy(gs_hbm.at[srcv.at[j + 1]], rows1, sem1).wait()
        pltpu.sync_copy(rows1, acc_sh.at[dstv.at[j + 1]], add=True)

        @pl.when(j + 3 < NCHUNK)
        def _():
            pltpu.async_copy(gs_hbm.at[srcv.at[j + 3]], rows1, sem1)

        return carry

    lax.fori_loop(0, NCHUNK // 2, body, 0)
    plsc.subcore_barrier()
    pltpu.sync_copy(acc_sh.at[pl.ds(s * ROWS_PT, ROWS_PT)], out_hbm.at[c, s])


def _tc1_body(degs_ref, x_ref, w1_ref, gs_ref, dinv_ref):
    deg = degs_ref[0][:, :1] + degs_ref[1][:, :1] + 1.0
    dinv = lax.rsqrt(deg)
    dinv_ref[...] = dinv
    h = jnp.dot(x_ref[...], w1_ref[...], preferred_element_type=jnp.float32)
    gs_ref[...] = h * dinv


_tc1 = pl.pallas_call(
    _tc1_body,
    out_shape=(
        jax.ShapeDtypeStruct((N_NODES, HID), jnp.float32),
        jax.ShapeDtypeStruct((N_NODES, 1), jnp.float32),
    ),
)


def _tc2_body(accp_ref, gs1_ref, dinv_ref, b1_ref, w2_ref, out_ref):
    agg = accp_ref[0] + accp_ref[1] + gs1_ref[...]
    h = jnp.maximum(agg * dinv_ref[...] + b1_ref[...], 0.0)
    out_ref[...] = jnp.dot(
        h, w2_ref[...], preferred_element_type=jnp.float32) * dinv_ref[...]


_tc2 = pl.pallas_call(
    _tc2_body,
    out_shape=jax.ShapeDtypeStruct((N_NODES, HID), jnp.float32),
)


def _tc3_body(accp_ref, gs2_ref, dinv_ref, b2_ref, wfc_ref, bfc_ref, out_ref):
    agg = accp_ref[0] + accp_ref[1] + gs2_ref[...]
    h = jnp.maximum(agg * dinv_ref[...] + b2_ref[...], 0.0)
    out_ref[...] = jnp.dot(
        h, wfc_ref[...], preferred_element_type=jnp.float32) + bfc_ref[...]


_tc3 = pl.pallas_call(
    _tc3_body,
    out_shape=jax.ShapeDtypeStruct((N_NODES, 1), jnp.float32),
)


def kernel(x, edge_index, W1, b1, W2, b2, W_fc, b_fc):
    src = edge_index[0].reshape(NW, NCHUNK, CHUNK)
    dst = edge_index[1].reshape(NW, NCHUNK, CHUNK)
    zrows = jnp.zeros((ZROWS, HID), jnp.float32)
    ones_deg = jnp.ones((CHUNK, DEGW), jnp.float32)
    zeros_deg = jnp.zeros((ROWS_PT, DEGW), jnp.float32)

    degp = _deg_kernel(dst, ones_deg, zeros_deg).reshape(NC, N_NODES, DEGW)
    gs1, dinv = _tc1(degp, x, W1)
    acc1 = _agg_kernel(gs1, src, dst, zrows).reshape(NC, N_NODES, HID)
    gs2 = _tc2(acc1, gs1, dinv, b1.reshape(1, HID), W2)
    acc2 = _agg_kernel(gs2, src, dst, zrows).reshape(NC, N_NODES, HID)
    out = _tc3(acc2, gs2, dinv, b2.reshape(1, HID), W_fc, b_fc.reshape(1, 1))
    return out


# speedup vs baseline: 39.6814x; 1.0374x over previous
import functools

import jax
import jax.numpy as jnp
from jax import lax
from jax.experimental import pallas as pl
from jax.experimental.pallas import tpu as pltpu
from jax.experimental.pallas import tpu_sc as plsc

N_NODES = 10000
N_EDGES = 320000
IN_CH = 128
HID = 64

NC = 2
NS = 16
NW = NC * NS
EPW = N_EDGES // NW
CHUNK = 125
NCHUNK = EPW // CHUNK
ROWS_PT = N_NODES // NS
ZROWS = 125

_mesh = plsc.VectorSubcoreMesh(core_axis_name="c", subcore_axis_name="s")
_sc_params = pltpu.CompilerParams(use_tc_tiling_on_sc=False)


DEGW = 16


@functools.partial(
    pl.kernel,
    out_type=jax.ShapeDtypeStruct((NC, N_NODES, DEGW), jnp.float32),
    mesh=_mesh,
    scratch_types=[
        pltpu.VMEM((NCHUNK, CHUNK), jnp.int32),
        pltpu.VMEM((CHUNK, DEGW), jnp.float32),
        pltpu.VMEM((ROWS_PT, DEGW), jnp.float32),
        pltpu.VMEM_SHARED((N_NODES, DEGW), jnp.float32),
    ],
    compiler_params=_sc_params,
)
def _deg_kernel(edges_hbm, ones_hbm, zeros_hbm, out_hbm, dstv, onesv, zbuf,
                deg_sh):
    c = lax.axis_index("c")
    s = lax.axis_index("s")
    pltpu.sync_copy(edges_hbm.at[1, c * NS + s], dstv)
    pltpu.sync_copy(ones_hbm, onesv)
    pltpu.sync_copy(zeros_hbm, zbuf)
    pltpu.sync_copy(zbuf, deg_sh.at[pl.ds(s * ROWS_PT, ROWS_PT)])
    plsc.subcore_barrier()

    def body(j, carry):
        pltpu.sync_copy(onesv, deg_sh.at[dstv.at[j]], add=True)
        return carry

    lax.fori_loop(0, NCHUNK, body, 0)
    plsc.subcore_barrier()
    pltpu.sync_copy(deg_sh.at[pl.ds(s * ROWS_PT, ROWS_PT)],
                    out_hbm.at[c, pl.ds(s * ROWS_PT, ROWS_PT)])


@functools.partial(
    pl.kernel,
    out_type=jax.ShapeDtypeStruct((NC, N_NODES, HID), jnp.float32),
    mesh=_mesh,
    scratch_types=[
        pltpu.VMEM((NCHUNK, CHUNK), jnp.int32),
        pltpu.VMEM((NCHUNK, CHUNK), jnp.int32),
        pltpu.VMEM((CHUNK, HID), jnp.float32),
        pltpu.VMEM((CHUNK, HID), jnp.float32),
        pltpu.VMEM((ZROWS, HID), jnp.float32),
        pltpu.VMEM_SHARED((N_NODES, HID), jnp.float32),
        pltpu.SemaphoreType.DMA,
        pltpu.SemaphoreType.DMA,
    ],
    compiler_params=_sc_params,
)
def _agg_kernel(gs_hbm, edges_hbm, zrows_hbm, out_hbm,
                srcv, dstv, rows0, rows1, zbuf, acc_sh, sem0, sem1):
    c = lax.axis_index("c")
    s = lax.axis_index("s")
    pltpu.sync_copy(edges_hbm.at[0, c * NS + s], srcv)
    pltpu.sync_copy(edges_hbm.at[1, c * NS + s], dstv)
    pltpu.sync_copy(zrows_hbm, zbuf)
    for t in range(ROWS_PT // ZROWS):
        pltpu.sync_copy(zbuf, acc_sh.at[pl.ds(s * ROWS_PT + t * ZROWS, ZROWS)])
    plsc.subcore_barrier()

    pltpu.async_copy(gs_hbm.at[srcv.at[0]], rows0, sem0)
    pltpu.async_copy(gs_hbm.at[srcv.at[1]], rows1, sem1)

    def body(i, carry):
        j = 2 * i
        pltpu.make_async_copy(gs_hbm.at[srcv.at[j]], rows0, sem0).wait()
        pltpu.sync_copy(rows0, acc_sh.at[dstv.at[j]], add=True)

        @pl.when(j + 2 < NCHUNK)
        def _():
            pltpu.async_copy(gs_hbm.at[srcv.at[j + 2]], rows0, sem0)

        pltpu.make_async_copy(gs_hbm.at[srcv.at[j + 1]], rows1, sem1).wait()
        pltpu.sync_copy(rows1, acc_sh.at[dstv.at[j + 1]], add=True)

        @pl.when(j + 3 < NCHUNK)
        def _():
            pltpu.async_copy(gs_hbm.at[srcv.at[j + 3]], rows1, sem1)

        return carry

    lax.fori_loop(0, NCHUNK // 2, body, 0)
    plsc.subcore_barrier()
    pltpu.sync_copy(acc_sh.at[pl.ds(s * ROWS_PT, ROWS_PT)],
                    out_hbm.at[c, pl.ds(s * ROWS_PT, ROWS_PT)])


def _tc1_body(degs_ref, x_ref, w1_ref, gs_ref, dinv_ref):
    deg = degs_ref[0][:, :1] + degs_ref[1][:, :1] + 1.0
    dinv = lax.rsqrt(deg)
    dinv_ref[...] = dinv
    h = jnp.dot(x_ref[...], w1_ref[...], preferred_element_type=jnp.float32)
    gs_ref[...] = h * dinv


_tc1 = pl.pallas_call(
    _tc1_body,
    out_shape=(
        jax.ShapeDtypeStruct((N_NODES, HID), jnp.float32),
        jax.ShapeDtypeStruct((N_NODES, 1), jnp.float32),
    ),
)


def _tc2_body(accp_ref, gs1_ref, dinv_ref, b1_ref, w2_ref, out_ref):
    agg = accp_ref[0] + accp_ref[1] + gs1_ref[...]
    h = jnp.maximum(agg * dinv_ref[...] + b1_ref[...], 0.0)
    out_ref[...] = jnp.dot(
        h, w2_ref[...], preferred_element_type=jnp.float32) * dinv_ref[...]


_tc2 = pl.pallas_call(
    _tc2_body,
    out_shape=jax.ShapeDtypeStruct((N_NODES, HID), jnp.float32),
)


def _tc3_body(accp_ref, gs2_ref, dinv_ref, b2_ref, wfc_ref, bfc_ref, out_ref):
    agg = accp_ref[0] + accp_ref[1] + gs2_ref[...]
    h = jnp.maximum(agg * dinv_ref[...] + b2_ref[...], 0.0)
    out_ref[...] = jnp.dot(
        h, wfc_ref[...], preferred_element_type=jnp.float32) + bfc_ref[...]


_tc3 = pl.pallas_call(
    _tc3_body,
    out_shape=jax.ShapeDtypeStruct((N_NODES, 1), jnp.float32),
)


def kernel(x, edge_index, W1, b1, W2, b2, W_fc, b_fc):
    edges = edge_index.reshape(2, NW, NCHUNK, CHUNK)
    zrows = jnp.zeros((ZROWS, HID), jnp.float32)
    ones_deg = jnp.ones((CHUNK, DEGW), jnp.float32)
    zeros_deg = jnp.zeros((ROWS_PT, DEGW), jnp.float32)

    degp = _deg_kernel(edges, ones_deg, zeros_deg)
    gs1, dinv = _tc1(degp, x, W1)
    acc1 = _agg_kernel(gs1, edges, zrows)
    gs2 = _tc2(acc1, gs1, dinv, b1.reshape(1, HID), W2)
    acc2 = _agg_kernel(gs2, edges, zrows)
    out = _tc3(acc2, gs2, dinv, b2.reshape(1, HID), W_fc, b_fc.reshape(1, 1))
    return out


# speedup vs baseline: 41.7707x; 1.0527x over previous
import functools

import jax
import jax.numpy as jnp
from jax import lax
from jax.experimental import pallas as pl
from jax.experimental.pallas import tpu as pltpu
from jax.experimental.pallas import tpu_sc as plsc

N_NODES = 10000
N_EDGES = 320000
IN_CH = 128
HID = 64

NC = 2
NS = 16
NW = NC * NS
EPW = N_EDGES // NW
CHUNK = 125
NCHUNK = EPW // CHUNK
ROWS_PT = N_NODES // NS
ZROWS = 125

_mesh = plsc.VectorSubcoreMesh(core_axis_name="c", subcore_axis_name="s")
_sc_params = pltpu.CompilerParams(use_tc_tiling_on_sc=False)


DEGW = 16


@functools.partial(
    pl.kernel,
    out_type=jax.ShapeDtypeStruct((NC, N_NODES, DEGW), jnp.float32),
    mesh=_mesh,
    scratch_types=[
        pltpu.VMEM((NCHUNK, CHUNK), jnp.int32),
        pltpu.VMEM((CHUNK, DEGW), jnp.float32),
        pltpu.VMEM((ROWS_PT, DEGW), jnp.float32),
        pltpu.VMEM_SHARED((N_NODES, DEGW), jnp.float32),
    ],
    compiler_params=_sc_params,
)
def _deg_kernel(edges_hbm, ones_hbm, zeros_hbm, out_hbm, dstv, onesv, zbuf,
                deg_sh):
    c = lax.axis_index("c")
    s = lax.axis_index("s")
    pltpu.sync_copy(edges_hbm.at[1, c * NS + s], dstv)
    pltpu.sync_copy(ones_hbm, onesv)
    pltpu.sync_copy(zeros_hbm, zbuf)
    pltpu.sync_copy(zbuf, deg_sh.at[pl.ds(s * ROWS_PT, ROWS_PT)])
    plsc.subcore_barrier()

    def body(j, carry):
        pltpu.sync_copy(onesv, deg_sh.at[dstv.at[j]], add=True)
        return carry

    lax.fori_loop(0, NCHUNK, body, 0)
    plsc.subcore_barrier()
    pltpu.sync_copy(deg_sh.at[pl.ds(s * ROWS_PT, ROWS_PT)],
                    out_hbm.at[c, pl.ds(s * ROWS_PT, ROWS_PT)])


@functools.partial(
    pl.kernel,
    out_type=jax.ShapeDtypeStruct((NC, N_NODES, HID), jnp.float32),
    mesh=_mesh,
    scratch_types=[
        pltpu.VMEM((NCHUNK, CHUNK), jnp.int32),
        pltpu.VMEM((NCHUNK, CHUNK), jnp.int32),
        [pltpu.VMEM((CHUNK, HID), jnp.float32) for _ in range(4)],
        pltpu.VMEM((ZROWS, HID), jnp.float32),
        pltpu.VMEM_SHARED((N_NODES, HID), jnp.float32),
        [pltpu.SemaphoreType.DMA for _ in range(4)],
        [pltpu.SemaphoreType.DMA for _ in range(4)],
    ],
    compiler_params=_sc_params,
)
def _agg_kernel(gs_hbm, edges_hbm, zrows_hbm, out_hbm,
                srcv, dstv, rows, zbuf, acc_sh, gsem, ssem):
    c = lax.axis_index("c")
    s = lax.axis_index("s")
    pltpu.sync_copy(edges_hbm.at[0, c * NS + s], srcv)
    pltpu.sync_copy(edges_hbm.at[1, c * NS + s], dstv)
    pltpu.sync_copy(zrows_hbm, zbuf)
    for t in range(ROWS_PT // ZROWS):
        pltpu.sync_copy(zbuf, acc_sh.at[pl.ds(s * ROWS_PT + t * ZROWS, ZROWS)])
    plsc.subcore_barrier()

    for b in range(4):
        pltpu.async_copy(gs_hbm.at[srcv.at[b]], rows[b], gsem[b])

    def body(i, carry):
        j = 4 * i
        for b in range(4):
            pltpu.make_async_copy(
                gs_hbm.at[srcv.at[j + b]], rows[b], gsem[b]).wait()
            pltpu.async_copy(
                rows[b], acc_sh.at[dstv.at[j + b]], ssem[b], add=True)
        for b in range(4):
            pltpu.make_async_copy(
                rows[b], acc_sh.at[dstv.at[j + b]], ssem[b]).wait()

            @pl.when(j + 4 + b < NCHUNK)
            def _():
                pltpu.async_copy(
                    gs_hbm.at[srcv.at[j + 4 + b]], rows[b], gsem[b])

        return carry

    lax.fori_loop(0, NCHUNK // 4, body, 0)
    plsc.subcore_barrier()
    pltpu.sync_copy(acc_sh.at[pl.ds(s * ROWS_PT, ROWS_PT)],
                    out_hbm.at[c, pl.ds(s * ROWS_PT, ROWS_PT)])


def _tc1_body(degs_ref, x_ref, w1_ref, gs_ref, dinv_ref):
    deg = degs_ref[0][:, :1] + degs_ref[1][:, :1] + 1.0
    dinv = lax.rsqrt(deg)
    dinv_ref[...] = dinv
    h = jnp.dot(x_ref[...], w1_ref[...], preferred_element_type=jnp.float32)
    gs_ref[...] = h * dinv


_tc1 = pl.pallas_call(
    _tc1_body,
    out_shape=(
        jax.ShapeDtypeStruct((N_NODES, HID), jnp.float32),
        jax.ShapeDtypeStruct((N_NODES, 1), jnp.float32),
    ),
)


def _tc2_body(accp_ref, gs1_ref, dinv_ref, b1_ref, w2_ref, out_ref):
    agg = accp_ref[0] + accp_ref[1] + gs1_ref[...]
    h = jnp.maximum(agg * dinv_ref[...] + b1_ref[...], 0.0)
    out_ref[...] = jnp.dot(
        h, w2_ref[...], preferred_element_type=jnp.float32) * dinv_ref[...]


_tc2 = pl.pallas_call(
    _tc2_body,
    out_shape=jax.ShapeDtypeStruct((N_NODES, HID), jnp.float32),
)


def _tc3_body(accp_ref, gs2_ref, dinv_ref, b2_ref, wfc_ref, bfc_ref, out_ref):
    agg = accp_ref[0] + accp_ref[1] + gs2_ref[...]
    h = jnp.maximum(agg * dinv_ref[...] + b2_ref[...], 0.0)
    out_ref[...] = jnp.dot(
        h, wfc_ref[...], preferred_element_type=jnp.float32) + bfc_ref[...]


_tc3 = pl.pallas_call(
    _tc3_body,
    out_shape=jax.ShapeDtypeStruct((N_NODES, 1), jnp.float32),
)


def kernel(x, edge_index, W1, b1, W2, b2, W_fc, b_fc):
    edges = edge_index.reshape(2, NW, NCHUNK, CHUNK)
    zrows = jnp.zeros((ZROWS, HID), jnp.float32)
    ones_deg = jnp.ones((CHUNK, DEGW), jnp.float32)
    zeros_deg = jnp.zeros((ROWS_PT, DEGW), jnp.float32)

    degp = _deg_kernel(edges, ones_deg, zeros_deg)
    gs1, dinv = _tc1(degp, x, W1)
    acc1 = _agg_kernel(gs1, edges, zrows)
    gs2 = _tc2(acc1, gs1, dinv, b1.reshape(1, HID), W2)
    acc2 = _agg_kernel(gs2, edges, zrows)
    out = _tc3(acc2, gs2, dinv, b2.reshape(1, HID), W_fc, b_fc.reshape(1, 1))
    return out


# speedup vs baseline: 45.7508x; 1.0953x over previous
import functools

import jax
import jax.numpy as jnp
from jax import lax
from jax.experimental import pallas as pl
from jax.experimental.pallas import tpu as pltpu
from jax.experimental.pallas import tpu_sc as plsc

N_NODES = 10000
N_EDGES = 320000
IN_CH = 128
HID = 64

NC = 2
NS = 16
NW = NC * NS
EPW = N_EDGES // NW
CHUNK = 125
NCHUNK = EPW // CHUNK
ROWS_PT = N_NODES // NS
ZROWS = 125

_mesh = plsc.VectorSubcoreMesh(core_axis_name="c", subcore_axis_name="s")
_sc_params = pltpu.CompilerParams(use_tc_tiling_on_sc=False)


DEGW = 16


@functools.partial(
    pl.kernel,
    out_type=jax.ShapeDtypeStruct((NC, N_NODES, 128), jnp.float32),
    mesh=_mesh,
    scratch_types=[
        pltpu.VMEM((NCHUNK, CHUNK), jnp.int32),
        pltpu.VMEM((CHUNK, DEGW), jnp.float32),
        pltpu.VMEM((ROWS_PT, DEGW), jnp.float32),
        pltpu.VMEM_SHARED((N_NODES, DEGW), jnp.float32),
    ],
    compiler_params=_sc_params,
)
def _deg_kernel(edges_hbm, ones_hbm, zeros_hbm, out_hbm, dstv, onesv, zbuf,
                deg_sh):
    c = lax.axis_index("c")
    s = lax.axis_index("s")
    pltpu.sync_copy(edges_hbm.at[1, c * NS + s], dstv)
    pltpu.sync_copy(ones_hbm, onesv)
    pltpu.sync_copy(zeros_hbm, zbuf)
    pltpu.sync_copy(zbuf, deg_sh.at[pl.ds(s * ROWS_PT, ROWS_PT)])
    plsc.subcore_barrier()

    def body(j, carry):
        pltpu.sync_copy(onesv, deg_sh.at[dstv.at[j]], add=True)
        return carry

    lax.fori_loop(0, NCHUNK, body, 0)
    plsc.subcore_barrier()
    pltpu.sync_copy(deg_sh.at[pl.ds(s * ROWS_PT, ROWS_PT)],
                    out_hbm.at[c, pl.ds(s * ROWS_PT, ROWS_PT), pl.ds(0, DEGW)])


@functools.partial(
    pl.kernel,
    out_type=jax.ShapeDtypeStruct((NC, N_NODES, 128), jnp.float32),
    mesh=_mesh,
    scratch_types=[
        pltpu.VMEM((NCHUNK, CHUNK), jnp.int32),
        pltpu.VMEM((NCHUNK, CHUNK), jnp.int32),
        [pltpu.VMEM((CHUNK, HID), jnp.float32) for _ in range(4)],
        pltpu.VMEM((ZROWS, HID), jnp.float32),
        pltpu.VMEM_SHARED((N_NODES, HID), jnp.float32),
        [pltpu.SemaphoreType.DMA for _ in range(4)],
        [pltpu.SemaphoreType.DMA for _ in range(4)],
    ],
    compiler_params=_sc_params,
)
def _agg_kernel(gs_hbm, edges_hbm, zrows_hbm, out_hbm,
                srcv, dstv, rows, zbuf, acc_sh, gsem, ssem):
    c = lax.axis_index("c")
    s = lax.axis_index("s")
    pltpu.sync_copy(edges_hbm.at[0, c * NS + s], srcv)
    pltpu.sync_copy(edges_hbm.at[1, c * NS + s], dstv)
    pltpu.sync_copy(zrows_hbm, zbuf)
    for t in range(ROWS_PT // ZROWS):
        pltpu.sync_copy(zbuf, acc_sh.at[pl.ds(s * ROWS_PT + t * ZROWS, ZROWS)])
    plsc.subcore_barrier()

    for b in range(4):
        pltpu.async_copy(gs_hbm.at[srcv.at[b]], rows[b], gsem[b])

    def body(i, carry):
        j = 4 * i
        for b in range(4):
            pltpu.make_async_copy(
                gs_hbm.at[srcv.at[j + b]], rows[b], gsem[b]).wait()
            pltpu.async_copy(
                rows[b], acc_sh.at[dstv.at[j + b]], ssem[b], add=True)
        for b in range(4):
            pltpu.make_async_copy(
                rows[b], acc_sh.at[dstv.at[j + b]], ssem[b]).wait()

            @pl.when(j + 4 + b < NCHUNK)
            def _():
                pltpu.async_copy(
                    gs_hbm.at[srcv.at[j + 4 + b]], rows[b], gsem[b])

        return carry

    lax.fori_loop(0, NCHUNK // 4, body, 0)
    plsc.subcore_barrier()
    pltpu.sync_copy(acc_sh.at[pl.ds(s * ROWS_PT, ROWS_PT)],
                    out_hbm.at[c, pl.ds(s * ROWS_PT, ROWS_PT), pl.ds(0, HID)])


def _tc1_body(degs_ref, x_ref, w1w_ref, gs_ref, dinv_ref):
    deg = degs_ref[0][:, :1] + degs_ref[1][:, :1] + 1.0
    dinv = lax.rsqrt(deg)
    dinv_ref[...] = dinv
    h = jnp.dot(x_ref[...], w1w_ref[...], preferred_element_type=jnp.float32)
    gs_ref[...] = h * dinv


_tc1 = pl.pallas_call(
    _tc1_body,
    out_shape=(
        jax.ShapeDtypeStruct((N_NODES, 128), jnp.float32),
        jax.ShapeDtypeStruct((N_NODES, 1), jnp.float32),
    ),
)


def _tc2_body(accw_ref, gs1_ref, dinv_ref, b1_ref, w2w_ref, out_ref):
    agg = accw_ref[0][:, :HID] + accw_ref[1][:, :HID] + gs1_ref[:, :HID]
    dinv = dinv_ref[...]
    h = jnp.maximum(agg * dinv + b1_ref[...], 0.0)
    out_ref[...] = jnp.dot(
        h, w2w_ref[...], preferred_element_type=jnp.float32) * dinv


_tc2 = pl.pallas_call(
    _tc2_body,
    out_shape=jax.ShapeDtypeStruct((N_NODES, 128), jnp.float32),
)


def _tc3_body(accw_ref, gs2_ref, dinv_ref, b2_ref, wfc_ref, bfc_ref, out_ref):
    agg = accw_ref[0][:, :HID] + accw_ref[1][:, :HID] + gs2_ref[:, :HID]
    h = jnp.maximum(agg * dinv_ref[...] + b2_ref[...], 0.0)
    out_ref[...] = jnp.dot(
        h, wfc_ref[...], preferred_element_type=jnp.float32) + bfc_ref[...]


_tc3 = pl.pallas_call(
    _tc3_body,
    out_shape=jax.ShapeDtypeStruct((N_NODES, 1), jnp.float32),
)


def kernel(x, edge_index, W1, b1, W2, b2, W_fc, b_fc):
    edges = jnp.stack([edge_index[0] * 2, edge_index[1]])
    edges = edges.reshape(2, NW, NCHUNK, CHUNK)
    zrows = jnp.zeros((ZROWS, HID), jnp.float32)
    ones_deg = jnp.ones((CHUNK, DEGW), jnp.float32)
    zeros_deg = jnp.zeros((ROWS_PT, DEGW), jnp.float32)
    zc = jnp.zeros((IN_CH, HID), jnp.float32)
    w1w = jnp.concatenate([W1, zc], axis=1)
    w2w = jnp.concatenate([W2, zc[:HID]], axis=1)

    degp = _deg_kernel(edges, ones_deg, zeros_deg)
    gs1, dinv = _tc1(degp, x, w1w)
    acc1 = _agg_kernel(gs1.reshape(2 * N_NODES, HID), edges, zrows)
    gs2 = _tc2(acc1, gs1, dinv, b1.reshape(1, HID), w2w)
    acc2 = _agg_kernel(gs2.reshape(2 * N_NODES, HID), edges, zrows)
    out = _tc3(acc2, gs2, dinv, b2.reshape(1, HID), W_fc, b_fc.reshape(1, 1))
    return out
